# baseline (speedup 1.0000x reference)
import functools

import jax
import jax.numpy as jnp
from jax.experimental import pallas as pl
from jax.experimental.pallas import tpu as pltpu
from jax.experimental.pallas import tpu_sc as plsc

_LT = 2048
_VT = 2048
_DP = 128


def _conv_pairs(x, pad=3):
    B, L, C = x.shape
    xp = jnp.pad(x, ((0, 0), (pad, pad), (0, 0)))
    return xp.reshape(B, (L + 2 * pad) // 2, 2 * C)


def _conv_mm_pairs(P, w8, b, relu, Lo):
    B, Lp, C2 = P.shape
    O = w8.shape[1]

    def body(p_ref, w_ref, b_ref, o_ref):
        x = p_ref[0]
        X = jnp.concatenate([x[j:j + Lo] for j in range(4)], axis=1)
        y = jax.lax.dot_general(
            X, w_ref[...], (((1,), (0,)), ((), ())),
            preferred_element_type=jnp.float32)
        y = y + b_ref[...]
        if relu:
            y = jnp.maximum(y, 0.0)
        o_ref[0] = y

    return pl.pallas_call(
        body,
        grid=(B,),
        in_specs=[
            pl.BlockSpec((1, Lp, C2), lambda bb: (bb, 0, 0)),
            pl.BlockSpec((4 * C2, O), lambda bb: (0, 0)),
            pl.BlockSpec((1, O), lambda bb: (0, 0)),
        ],
        out_specs=pl.BlockSpec((1, Lo, O), lambda bb: (bb, 0, 0)),
        out_shape=jax.ShapeDtypeStruct((B, Lo, O), jnp.float32),
    )(P, w8, b.reshape(1, O))


def _conv_mm(h, w8, b, relu):
    B, L, C = h.shape
    Lo = L // 2
    O = w8.shape[1]

    def body(h_ref, w_ref, b_ref, o_ref):
        x = h_ref[0]
        xp = jnp.pad(x, ((3, 5), (0, 0)))
        ph = xp.reshape((L + 8) // 2, 2, C)
        xe = ph[:, 0, :]
        xo = ph[:, 1, :]
        taps = []
        for j in range(4):
            taps.append(xe[j:j + Lo])
            taps.append(xo[j:j + Lo])
        X = jnp.concatenate(taps, axis=1)
        y = jax.lax.dot_general(
            X, w_ref[...], (((1,), (0,)), ((), ())),
            preferred_element_type=jnp.float32)
        y = y + b_ref[...]
        if relu:
            y = jnp.maximum(y, 0.0)
        o_ref[0] = y

    return pl.pallas_call(
        body,
        grid=(B,),
        in_specs=[
            pl.BlockSpec((1, L, C), lambda bb: (bb, 0, 0)),
            pl.BlockSpec((8 * C, O), lambda bb: (0, 0)),
            pl.BlockSpec((1, O), lambda bb: (0, 0)),
        ],
        out_specs=pl.BlockSpec((1, Lo, O), lambda bb: (bb, 0, 0)),
        out_shape=jax.ShapeDtypeStruct((B, Lo, O), jnp.float32),
    )(h, w8, b.reshape(1, O))


def _w8(w):
    O, C, _ = w.shape
    w7 = jnp.transpose(w, (2, 1, 0)).reshape(7 * C, O)
    return jnp.concatenate([w7, jnp.zeros((C, O), w7.dtype)], axis=0)


def _conv_stack(audio, w1, b1, w2, b2, w3, b3):
    B = audio.shape[0]
    L = audio.shape[2]
    x = jnp.reshape(audio, (B, L, 1))
    h1 = _conv_mm_pairs(_conv_pairs(x), _w8(w1), b1, True, L // 2)
    h2 = _conv_mm(h1, _w8(w2), b2, True)
    f = _conv_mm(h2, _w8(w3), b3, False)
    return f.reshape(B * (L // 8), -1)


def _vq_tokens_one(f, cbT):
    BL, D = f.shape
    V = cbT.shape[1]
    nl = BL // _LT
    nv = V // _VT

    def body(f_ref, cb_ref, tok_ref, minv, argm):
        j = pl.program_id(1)

        @pl.when(j == 0)
        def _init():
            minv[...] = jnp.full(minv.shape, jnp.inf, jnp.float32)
            argm[...] = jnp.zeros(argm.shape, jnp.int32)

        fv = f_ref[...]
        cb = cb_ref[...]
        f2 = jnp.sum(fv * fv, axis=1, keepdims=True)
        c2 = jnp.sum(cb * cb, axis=0, keepdims=True)
        e2 = jax.lax.dot_general(
            fv, cb * -2.0, (((1,), (0,)), ((), ())),
            preferred_element_type=jnp.float32)
        CW = 128
        m = None
        a = None
        for c in range(_VT // CW):
            sl = slice(c * CW, (c + 1) * CW)
            s = (f2 + c2[:, sl]) + e2[:, sl]
            if c == 0:
                m = s
                a = jnp.zeros(s.shape, jnp.float32)
            else:
                upd = s < m
                a = jnp.where(upd, jnp.float32(c), a)
                m = jnp.minimum(m, s)
        tmin = jnp.min(m, axis=1, keepdims=True)
        lane = jax.lax.broadcasted_iota(
            jnp.int32, m.shape, 1).astype(jnp.float32)
        gidx = a * jnp.float32(CW) + lane
        targ_f = jnp.min(jnp.where(m == tmin, gidx, jnp.float32(_VT)),
                         axis=1, keepdims=True)
        targ = targ_f.astype(jnp.int32) + j * _VT
        better = tmin < minv[...]
        argm[...] = jnp.where(better, targ, argm[...])
        minv[...] = jnp.where(better, tmin, minv[...])

        @pl.when(j == nv - 1)
        def _fin():
            tok_ref[...] = argm[...]

    return pl.pallas_call(
        body,
        grid=(nl, nv),
        in_specs=[
            pl.BlockSpec((_LT, D), lambda l, j: (l, 0)),
            pl.BlockSpec((D, _VT), lambda l, j: (0, j)),
        ],
        out_specs=pl.BlockSpec((_LT, 1), lambda l, j: (l, 0)),
        out_shape=jax.ShapeDtypeStruct((BL, 1), jnp.int32),
        scratch_shapes=[
            pltpu.VMEM((_LT, 1), jnp.float32),
            pltpu.VMEM((_LT, 1), jnp.int32),
        ],
    )(f, cbT)


def _sc_gather_one(tokens, emb_pad):
    BL = tokens.shape[0]
    V = emb_pad.shape[0]
    NC, NS = 2, 16
    NW = NC * NS
    CHUNK = BL // NW
    VS = V // NS
    mesh = plsc.VectorSubcoreMesh(core_axis_name="c", subcore_axis_name="s")

    @functools.partial(
        pl.kernel, mesh=mesh,
        out_type=jax.ShapeDtypeStruct((BL, _DP), jnp.float32),
        scratch_types=[
            pltpu.VMEM((CHUNK,), jnp.int32),
            pltpu.VMEM((CHUNK, _DP), jnp.float32),
            pltpu.VMEM_SHARED((V, _DP), jnp.float32),
            pltpu.SemaphoreType.DMA,
        ],
    )
    def gather_kernel(tok_hbm, emb_hbm, out_hbm, idx_v, rows_v, shared, sem):
        sid = jax.lax.axis_index("s")
        wid = sid * NC + jax.lax.axis_index("c")
        base = wid * CHUNK
        pltpu.sync_copy(emb_hbm.at[pl.ds(sid * VS, VS)],
                        shared.at[pl.ds(sid * VS, VS)])
        pltpu.sync_copy(tok_hbm.at[pl.ds(base, CHUNK)], idx_v)
        plsc.subcore_barrier()
        pltpu.async_copy(shared.at[idx_v], rows_v, sem).wait()
        pltpu.sync_copy(rows_v, out_hbm.at[pl.ds(base, CHUNK)])

    return gather_kernel(tokens, emb_pad)


def _mean_kernel(parts, D):
    BL = parts[0].shape[0]
    ncb = len(parts)
    scale = 1.0 / ncb

    def body(*refs):
        o_ref = refs[-1]
        s = refs[0][...]
        for r in refs[1:-1]:
            s = s + r[...]
        o_ref[...] = s[:, :D] * scale

    return pl.pallas_call(
        body,
        out_shape=jax.ShapeDtypeStruct((BL, D), jnp.float32),
    )(*parts)


def kernel(audio, w1, b1, w2, b2, w3, b3, codebook, emb_table):
    B = audio.shape[0]
    ncb = codebook.shape[0]
    D = emb_table.shape[1]
    f = _conv_stack(audio, w1, b1, w2, b2, w3, b3)
    cbT = jnp.transpose(codebook, (0, 2, 1))
    emb_pad = jnp.pad(emb_table, ((0, 0), (0, _DP - D)))
    toks = []
    parts = []
    for i in range(ncb):
        t = _vq_tokens_one(f, cbT[i])[:, 0]
        toks.append(t)
        parts.append(_sc_gather_one(t, emb_pad))
    emb = _mean_kernel(parts, D)
    BL = f.shape[0]
    L3 = BL // B
    tokens = jnp.transpose(jnp.stack(toks, 0).reshape(ncb, B, L3), (1, 0, 2))
    embeddings = emb.reshape(B, L3, D)
    return tokens, embeddings


# speedup vs baseline: 1.0420x; 1.0420x over previous
import functools

import jax
import jax.numpy as jnp
from jax.experimental import pallas as pl
from jax.experimental.pallas import tpu as pltpu
from jax.experimental.pallas import tpu_sc as plsc

_LT = 4096
_VT = 2048
_DP = 128


def _conv_pairs(x, pad=3):
    B, L, C = x.shape
    xp = jnp.pad(x, ((0, 0), (pad, pad), (0, 0)))
    return xp.reshape(B, (L + 2 * pad) // 2, 2 * C)


def _conv_mm_pairs(P, w8, b, relu, Lo):
    B, Lp, C2 = P.shape
    O = w8.shape[1]

    def body(p_ref, w_ref, b_ref, o_ref):
        x = p_ref[0]
        X = jnp.concatenate([x[j:j + Lo] for j in range(4)], axis=1)
        y = jax.lax.dot_general(
            X, w_ref[...], (((1,), (0,)), ((), ())),
            preferred_element_type=jnp.float32)
        y = y + b_ref[...]
        if relu:
            y = jnp.maximum(y, 0.0)
        o_ref[0] = y

    return pl.pallas_call(
        body,
        grid=(B,),
        in_specs=[
            pl.BlockSpec((1, Lp, C2), lambda bb: (bb, 0, 0)),
            pl.BlockSpec((4 * C2, O), lambda bb: (0, 0)),
            pl.BlockSpec((1, O), lambda bb: (0, 0)),
        ],
        out_specs=pl.BlockSpec((1, Lo, O), lambda bb: (bb, 0, 0)),
        out_shape=jax.ShapeDtypeStruct((B, Lo, O), jnp.float32),
    )(P, w8, b.reshape(1, O))


def _w8(w):
    O, C, _ = w.shape
    w7 = jnp.transpose(w, (2, 1, 0)).reshape(7 * C, O)
    return jnp.concatenate([w7, jnp.zeros((C, O), w7.dtype)], axis=0)


def _conv_stack(audio, w1, b1, w2, b2, w3, b3):
    B = audio.shape[0]
    L = audio.shape[2]
    x = jnp.reshape(audio, (B, L, 1))
    h1 = _conv_mm_pairs(_conv_pairs(x), _w8(w1), b1, True, L // 2)
    h2 = _conv_mm_pairs(_conv_pairs(h1), _w8(w2), b2, True, L // 4)
    f = _conv_mm_pairs(_conv_pairs(h2), _w8(w3), b3, False, L // 8)
    return f.reshape(B * (L // 8), -1)


def _vq_tokens_one(f, cbT):
    BL, D = f.shape
    V = cbT.shape[1]
    nl = BL // _LT
    nv = V // _VT

    def body(f_ref, cb_ref, tok_ref, minv, argm):
        j = pl.program_id(1)

        @pl.when(j == 0)
        def _init():
            minv[...] = jnp.full(minv.shape, jnp.inf, jnp.float32)
            argm[...] = jnp.zeros(argm.shape, jnp.int32)

        fv = f_ref[...]
        cb = cb_ref[...]
        f2 = jnp.sum(fv * fv, axis=1, keepdims=True)
        c2 = jnp.sum(cb * cb, axis=0, keepdims=True)
        e2 = jax.lax.dot_general(
            fv, cb * -2.0, (((1,), (0,)), ((), ())),
            preferred_element_type=jnp.float32)
        CW = 128
        m = None
        a = None
        for c in range(_VT // CW):
            sl = slice(c * CW, (c + 1) * CW)
            s = (f2 + c2[:, sl]) + e2[:, sl]
            if c == 0:
                m = s
                a = jnp.zeros(s.shape, jnp.float32)
            else:
                upd = s < m
                a = jnp.where(upd, jnp.float32(c), a)
                m = jnp.minimum(m, s)
        tmin = jnp.min(m, axis=1, keepdims=True)
        lane = jax.lax.broadcasted_iota(
            jnp.int32, m.shape, 1).astype(jnp.float32)
        gidx = a * jnp.float32(CW) + lane
        targ_f = jnp.min(jnp.where(m == tmin, gidx, jnp.float32(_VT)),
                         axis=1, keepdims=True)
        targ = targ_f.astype(jnp.int32) + j * _VT
        better = tmin < minv[...]
        argm[...] = jnp.where(better, targ, argm[...])
        minv[...] = jnp.where(better, tmin, minv[...])

        @pl.when(j == nv - 1)
        def _fin():
            tok_ref[...] = argm[...]

    return pl.pallas_call(
        body,
        grid=(nl, nv),
        in_specs=[
            pl.BlockSpec((_LT, D), lambda l, j: (l, 0)),
            pl.BlockSpec((D, _VT), lambda l, j: (0, j)),
        ],
        out_specs=pl.BlockSpec((_LT, 1), lambda l, j: (l, 0)),
        out_shape=jax.ShapeDtypeStruct((BL, 1), jnp.int32),
        scratch_shapes=[
            pltpu.VMEM((_LT, 1), jnp.float32),
            pltpu.VMEM((_LT, 1), jnp.int32),
        ],
    )(f, cbT)


def _sc_gather_one(tokens, emb_pad):
    BL = tokens.shape[0]
    V = emb_pad.shape[0]
    NC, NS = 2, 16
    NW = NC * NS
    CHUNK = BL // NW
    VS = V // NS
    mesh = plsc.VectorSubcoreMesh(core_axis_name="c", subcore_axis_name="s")

    @functools.partial(
        pl.kernel, mesh=mesh,
        out_type=jax.ShapeDtypeStruct((BL, _DP), jnp.float32),
        scratch_types=[
            pltpu.VMEM((CHUNK,), jnp.int32),
            pltpu.VMEM((CHUNK, _DP), jnp.float32),
            pltpu.VMEM_SHARED((V, _DP), jnp.float32),
            pltpu.SemaphoreType.DMA,
        ],
    )
    def gather_kernel(tok_hbm, emb_hbm, out_hbm, idx_v, rows_v, shared, sem):
        sid = jax.lax.axis_index("s")
        wid = sid * NC + jax.lax.axis_index("c")
        base = wid * CHUNK
        pltpu.sync_copy(emb_hbm.at[pl.ds(sid * VS, VS)],
                        shared.at[pl.ds(sid * VS, VS)])
        pltpu.sync_copy(tok_hbm.at[pl.ds(base, CHUNK)], idx_v)
        plsc.subcore_barrier()
        pltpu.async_copy(shared.at[idx_v], rows_v, sem).wait()
        pltpu.sync_copy(rows_v, out_hbm.at[pl.ds(base, CHUNK)])

    return gather_kernel(tokens, emb_pad)


def _mean_kernel(parts, D):
    BL = parts[0].shape[0]
    ncb = len(parts)
    scale = 1.0 / ncb

    def body(*refs):
        o_ref = refs[-1]
        s = refs[0][...]
        for r in refs[1:-1]:
            s = s + r[...]
        o_ref[...] = s[:, :D] * scale

    return pl.pallas_call(
        body,
        out_shape=jax.ShapeDtypeStruct((BL, D), jnp.float32),
    )(*parts)


def kernel(audio, w1, b1, w2, b2, w3, b3, codebook, emb_table):
    B = audio.shape[0]
    ncb = codebook.shape[0]
    D = emb_table.shape[1]
    f = _conv_stack(audio, w1, b1, w2, b2, w3, b3)
    cbT = jnp.transpose(codebook, (0, 2, 1))
    emb_pad = jnp.pad(emb_table, ((0, 0), (0, _DP - D)))
    toks = []
    parts = []
    for i in range(ncb):
        t = _vq_tokens_one(f, cbT[i])[:, 0]
        toks.append(t)
        parts.append(_sc_gather_one(t, emb_pad))
    emb = _mean_kernel(parts, D)
    BL = f.shape[0]
    L3 = BL // B
    tokens = jnp.transpose(jnp.stack(toks, 0).reshape(ncb, B, L3), (1, 0, 2))
    embeddings = emb.reshape(B, L3, D)
    return tokens, embeddings
